# Initial kernel scaffold; baseline (speedup 1.0000x reference)
import jax
import jax.numpy as jnp
from jax.experimental import pallas as pl


def kernel(logits, targets):
    raise NotImplementedError("write your pallas kernel here")


# speedup vs baseline: 30.8050x; 30.8050x over previous
import functools

import jax
import jax.numpy as jnp
from jax import lax
from jax.experimental import pallas as pl
from jax.experimental.pallas import tpu as pltpu
from jax.experimental.pallas import tpu_sc as plsc

N = 8388608
K_SAMPLES = max(1, int(N * 0.2))
NB = 32768
NC, NS = 2, 16
NW = NC * NS
PER_W = N // NW
CHUNK = 16384
NCHUNK = PER_W // CHUNK
ROWS1, COLS1 = 8192, 1024
HR, HC = 256, 128


def _bce_body(x_ref, t_ref, o_ref):
    x = x_ref[...]
    t = t_ref[...]
    o_ref[...] = jnp.maximum(x, 0.0) - x * t + jnp.log1p(jnp.exp(-jnp.abs(x)))


def _bce(x2, t2):
    grid = 16
    rows = ROWS1 // grid
    return pl.pallas_call(
        _bce_body,
        grid=(grid,),
        in_specs=[pl.BlockSpec((rows, COLS1), lambda i: (i, 0))] * 2,
        out_specs=pl.BlockSpec((rows, COLS1), lambda i: (i, 0)),
        out_shape=jax.ShapeDtypeStruct((ROWS1, COLS1), jnp.float32),
    )(x2, t2)


def _hist_body(bce_hbm, cnt_hbm, sum_hbm, buf0, buf1, hc, hs, sem0, sem1):
    cid = lax.axis_index("c")
    sid = lax.axis_index("s")
    wid = sid * NC + cid
    base = wid * PER_W

    zeros16 = jnp.zeros((16,), jnp.float32)

    @pl.loop(0, NB // 16, unroll=4)
    def _zero(i):
        hc[pl.ds(i * 16, 16)] = zeros16
        hs[pl.ds(i * 16, 16)] = zeros16

    bufs = (buf0, buf1)
    sems = (sem0, sem1)
    ones16 = jnp.ones((16,), jnp.float32)

    descs = [None] * NCHUNK
    descs[0] = pltpu.async_copy(bce_hbm.at[pl.ds(base, CHUNK)], bufs[0], sems[0])
    for c in range(NCHUNK):
        if c + 1 < NCHUNK:
            descs[c + 1] = pltpu.async_copy(
                bce_hbm.at[pl.ds(base + (c + 1) * CHUNK, CHUNK)],
                bufs[(c + 1) % 2],
                sems[(c + 1) % 2],
            )
        descs[c].wait()
        buf = bufs[c % 2]

        @pl.loop(0, CHUNK // 16, unroll=4)
        def _proc(j):
            v = buf[pl.ds(j * 16, 16)]
            bits = lax.bitcast_convert_type(v, jnp.int32)
            idx = jnp.minimum(jnp.maximum(bits >> 16, 0), NB - 1)
            plsc.addupdate_scatter(hc, [idx], ones16)
            plsc.addupdate_scatter(hs, [idx], v)

    pltpu.sync_copy(hc, cnt_hbm.at[wid])
    pltpu.sync_copy(hs, sum_hbm.at[wid])


def _hist(bce_flat):
    f32 = jnp.float32
    return pl.kernel(
        _hist_body,
        out_type=(
            jax.ShapeDtypeStruct((NW, NB), f32),
            jax.ShapeDtypeStruct((NW, NB), f32),
        ),
        mesh=plsc.VectorSubcoreMesh(core_axis_name="c", subcore_axis_name="s"),
        compiler_params=pltpu.CompilerParams(needs_layout_passes=False),
        scratch_types=[
            pltpu.VMEM((CHUNK,), f32),
            pltpu.VMEM((CHUNK,), f32),
            pltpu.VMEM((NB,), f32),
            pltpu.VMEM((NB,), f32),
            pltpu.SemaphoreType.DMA,
            pltpu.SemaphoreType.DMA,
        ],
    )(bce_flat)


def _final_body(cnt_ref, sum_ref, o_ref):
    f32 = jnp.float32
    c = jnp.sum(cnt_ref[...], axis=0)
    s = jnp.sum(sum_ref[...], axis=0)

    a128 = lax.broadcasted_iota(jnp.int32, (HC, HC), 0)
    b128 = lax.broadcasted_iota(jnp.int32, (HC, HC), 1)
    m1 = (a128 >= b128).astype(f32)
    inrow = lax.dot(c, m1, precision=lax.Precision.HIGHEST,
                    preferred_element_type=f32)
    rs = jnp.sum(c, axis=1, keepdims=True)
    a256 = lax.broadcasted_iota(jnp.int32, (HR, HR), 0)
    b256 = lax.broadcasted_iota(jnp.int32, (HR, HR), 1)
    g = (b256 > a256).astype(f32)
    above = lax.dot(g, rs, precision=lax.Precision.HIGHEST,
                    preferred_element_type=f32)
    suffix = inrow + above

    kf = f32(K_SAMPLES)
    fidx = (
        lax.broadcasted_iota(jnp.int32, (HR, HC), 0) * HC
        + lax.broadcasted_iota(jnp.int32, (HR, HC), 1)
    )
    bstar = jnp.max(jnp.where(suffix >= kf, fidx, -1))
    onehot = (fidx == bstar).astype(f32)
    c_b = jnp.sum(onehot * c)
    s_b = jnp.sum(onehot * s)
    suf_b = jnp.sum(onehot * suffix)
    count_above = suf_b - c_b
    sum_above = jnp.sum(jnp.where(fidx > bstar, s, f32(0.0)))
    m = kf - count_above
    o_ref[0, 0] = (sum_above + m * (s_b / c_b)) / kf


def _finalize(cnt3, sum3):
    return pl.pallas_call(
        _final_body,
        out_shape=jax.ShapeDtypeStruct((1, 1), jnp.float32),
        out_specs=pl.BlockSpec(memory_space=pltpu.SMEM),
    )(cnt3, sum3)


@jax.jit
def kernel(logits, targets):
    x2 = logits.reshape(ROWS1, COLS1)
    t2 = targets.reshape(ROWS1, COLS1)
    bce = _bce(x2, t2)
    cnt, sm = _hist(bce.reshape(N))
    out = _finalize(cnt.reshape(NW, HR, HC), sm.reshape(NW, HR, HC))
    return out.reshape(())
